# baseline (speedup 1.0000x reference)
import jax
import jax.numpy as jnp
from jax.experimental import pallas as pl


def kernel(partial_charge, atomic_number, degree, ring_encoding, edge_index,
           W_in, b_in, W_att, a_src, a_dst, W1, b1, W_out, b_out):
    n = partial_charge.shape[0]
    x = jnp.concatenate([partial_charge, atomic_number, degree, ring_encoding], axis=-1)
    x = jax.nn.elu(x @ W_in + b_in)
    src = edge_index[0]
    dst = edge_index[1]
    h = x @ W_att
    alpha_s = h @ a_src
    alpha_d = h @ a_dst
    e = jax.nn.leaky_relu(alpha_s[src] + alpha_d[dst], negative_slope=0.2)
    m = jax.lax.stop_gradient(jax.ops.segment_max(e, dst, num_segments=n))
    m = jnp.where(jnp.isfinite(m), m, 0.0)
    e_exp = jnp.exp(e - m[dst])
    denom = jax.ops.segment_sum(e_exp, dst, num_segments=n)
    w = e_exp / (denom[dst] + 1e-16)
    agg = jax.ops.segment_sum(w[:, None] * h[src], dst, num_segments=n)
    h_node = x + agg
    s = jax.nn.elu(h_node)
    s = s @ W1 + b1
    s = jax.nn.elu(s)
    s = s @ W_out + b_out
    return s


# speedup vs baseline: 28.9106x; 28.9106x over previous
import jax
import jax.numpy as jnp
from jax import lax
from jax.experimental import pallas as pl
from jax.experimental.pallas import tpu as pltpu
from jax.experimental.pallas import tpu_sc as plsc

N = 100000
E = 1600000
H = 32
N_PAD = 102400
HALF = N_PAD // 2
BLK = 2048
GRID = N_PAD // BLK

NTILE = 16
EDGES_PER_TILE = E // NTILE
CHUNK = 400
NCHUNK = EDGES_PER_TILE // CHUNK
VREGS = CHUNK // 16
SEG = N_PAD // NTILE
ASTG = 640
DRAIN = HALF // NTILE


def _pre_body(xin_ref, win_ref, bin_ref, watt_ref, asr_ref, adr_ref,
              x_ref, h_ref, al_ref):
    z = jnp.dot(xin_ref[...], win_ref[...],
                preferred_element_type=jnp.float32) + bin_ref[...]
    x = jnp.where(z > 0, z, jnp.exp(z) - 1.0)
    x_ref[...] = x
    h = jnp.dot(x, watt_ref[...], preferred_element_type=jnp.float32)
    h_ref[...] = h
    a_s = lax.dot_general(asr_ref[...], h, (((1,), (1,)), ((), ())),
                          preferred_element_type=jnp.float32)
    a_d = lax.dot_general(adr_ref[...], h, (((1,), (1,)), ((), ())),
                          preferred_element_type=jnp.float32)
    al_ref[...] = jnp.concatenate(
        [jnp.broadcast_to(a_s, (8, BLK)), jnp.broadcast_to(a_d, (8, BLK))], 0)


_pre_call = pl.pallas_call(
    _pre_body,
    grid=(GRID,),
    in_specs=[
        pl.BlockSpec((BLK, 48), lambda i: (i, 0)),
        pl.BlockSpec((48, H), lambda i: (0, 0)),
        pl.BlockSpec((1, H), lambda i: (0, 0)),
        pl.BlockSpec((H, H), lambda i: (0, 0)),
        pl.BlockSpec((1, H), lambda i: (0, 0)),
        pl.BlockSpec((1, H), lambda i: (0, 0)),
    ],
    out_specs=[
        pl.BlockSpec((BLK, H), lambda i: (i, 0)),
        pl.BlockSpec((BLK, H), lambda i: (i, 0)),
        pl.BlockSpec((16, BLK), lambda i: (0, i)),
    ],
    out_shape=[
        jax.ShapeDtypeStruct((N_PAD, H), jnp.float32),
        jax.ShapeDtypeStruct((N_PAD, H), jnp.float32),
        jax.ShapeDtypeStruct((16, N_PAD), jnp.float32),
    ],
)


def _edge_body(h_hbm, al_hbm, src_hbm, dst_hbm,
               agg_hbm, den_hbm, as1_hbm, ad1_hbm,
               agg_sp, den_sp,
               srcv, dstv, asv, adv, pbuf, dlocv, rows, abuf, sem):
    c = lax.axis_index("c")
    s = lax.axis_index("s")
    lo = c * HALF

    z16 = jnp.zeros((16,), jnp.float32)

    def zr_body(i, _):
        rows[i, pl.ds(0, 16)] = z16
        rows[i, pl.ds(16, 16)] = z16
        return 0

    lax.fori_loop(0, CHUNK, zr_body, 0)

    def zp_body(i, _):
        pbuf[pl.ds(i * 16, 16)] = z16
        return 0

    lax.fori_loop(0, VREGS, zp_body, 0)

    db = s * DRAIN
    for q in range(DRAIN // CHUNK):
        pltpu.sync_copy(rows, agg_sp.at[pl.ds(db + q * CHUNK, CHUNK)])
        pltpu.sync_copy(pbuf, den_sp.at[pl.ds(db + q * CHUNK, CHUNK)])

    @pl.when(s == 0)
    def _zero_trash():
        pltpu.sync_copy(rows.at[pl.ds(0, 16)], agg_sp.at[pl.ds(HALF, 16)])
        pltpu.sync_copy(pbuf.at[pl.ds(0, 16)], den_sp.at[pl.ds(HALF, 16)])

    def al_body(q, _):
        b2 = s * SEG + q * ASTG
        pltpu.sync_copy(al_hbm.at[pl.ds(0, 8), pl.ds(b2, ASTG)], abuf)
        pltpu.sync_copy(abuf.at[0], as1_hbm.at[pl.ds(b2, ASTG)])
        pltpu.sync_copy(al_hbm.at[pl.ds(8, 8), pl.ds(b2, ASTG)], abuf)
        pltpu.sync_copy(abuf.at[0], ad1_hbm.at[pl.ds(b2, ASTG)])
        return 0

    lax.fori_loop(0, SEG // ASTG, al_body, 0)

    plsc.subcore_barrier()

    lane = lax.iota(jnp.int32, 16)

    def chunk_body(k, _):
        base = s * EDGES_PER_TILE + k * CHUNK
        pltpu.sync_copy(src_hbm.at[pl.ds(base, CHUNK)], srcv)
        pltpu.sync_copy(dst_hbm.at[pl.ds(base, CHUNK)], dstv)
        pltpu.async_copy(as1_hbm.at[srcv], asv, sem).wait()
        pltpu.async_copy(ad1_hbm.at[dstv], adv, sem).wait()
        pltpu.async_copy(h_hbm.at[srcv], rows, sem).wait()

        def vreg_body(j, _):
            o = j * 16
            a = asv[pl.ds(o, 16)] + adv[pl.ds(o, 16)]
            e = jnp.maximum(a, 0.2 * a)
            p = jnp.exp(e)
            dd = dstv[pl.ds(o, 16)]
            inhalf = (dd >= lo) & (dd < lo + HALF)
            pbuf[pl.ds(o, 16)] = jnp.where(inhalf, p, 0.0)
            dlocv[pl.ds(o, 16)] = jnp.where(inhalf, dd - lo, HALF + lane)
            return 0

        lax.fori_loop(0, VREGS, vreg_body, 0)

        def wbody(g, _):
            o = g * 16
            pv16 = pbuf[pl.ds(o, 16)]
            for u in range(16):
                ei = o + u
                pv = jnp.full((16,), pv16[u], jnp.float32)
                rows[ei, pl.ds(0, 16)] = rows[ei, pl.ds(0, 16)] * pv
                rows[ei, pl.ds(16, 16)] = rows[ei, pl.ds(16, 16)] * pv
            return 0

        lax.fori_loop(0, VREGS, wbody, 0)

        pltpu.sync_copy(rows, agg_sp.at[dlocv], add=True)
        pltpu.sync_copy(pbuf, den_sp.at[dlocv], add=True)
        return 0

    lax.fori_loop(0, NCHUNK, chunk_body, 0)

    plsc.subcore_barrier()

    ob = lo + db
    for q in range(DRAIN // CHUNK):
        qq = q * CHUNK
        pltpu.sync_copy(agg_sp.at[pl.ds(db + qq, CHUNK)], rows)
        pltpu.sync_copy(rows, agg_hbm.at[pl.ds(ob + qq, CHUNK)])
        pltpu.sync_copy(den_sp.at[pl.ds(db + qq, CHUNK)], pbuf)
        pltpu.sync_copy(pbuf, den_hbm.at[pl.ds(ob + qq, CHUNK)])


_edge_call = pl.kernel(
    _edge_body,
    out_type=[
        jax.ShapeDtypeStruct((N_PAD, H), jnp.float32),
        jax.ShapeDtypeStruct((N_PAD,), jnp.float32),
        jax.ShapeDtypeStruct((N_PAD,), jnp.float32),
        jax.ShapeDtypeStruct((N_PAD,), jnp.float32),
    ],
    mesh=plsc.VectorSubcoreMesh(core_axis_name="c", subcore_axis_name="s"),
    compiler_params=pltpu.CompilerParams(use_tc_tiling_on_sc=False),
    scratch_types=[
        pltpu.VMEM_SHARED((HALF + 16, H), jnp.float32),
        pltpu.VMEM_SHARED((HALF + 16,), jnp.float32),
        pltpu.VMEM((CHUNK,), jnp.int32),
        pltpu.VMEM((CHUNK,), jnp.int32),
        pltpu.VMEM((CHUNK,), jnp.float32),
        pltpu.VMEM((CHUNK,), jnp.float32),
        pltpu.VMEM((CHUNK,), jnp.float32),
        pltpu.VMEM((CHUNK,), jnp.int32),
        pltpu.VMEM((CHUNK, H), jnp.float32),
        pltpu.VMEM((8, ASTG), jnp.float32),
        pltpu.SemaphoreType.DMA,
    ],
)


def _head_body(x_ref, agg_ref, den_ref, w1_ref, b1_ref, wo_ref, bo_ref, o_ref):
    hn = x_ref[...] + agg_ref[...] / (den_ref[...] + 1e-16)
    sa = jnp.where(hn > 0, hn, jnp.exp(hn) - 1.0)
    sb = jnp.dot(sa, w1_ref[...], preferred_element_type=jnp.float32) + b1_ref[...]
    sc = jnp.where(sb > 0, sb, jnp.exp(sb) - 1.0)
    o_ref[...] = jnp.dot(sc, wo_ref[...],
                         preferred_element_type=jnp.float32) + bo_ref[...]


_head_call = pl.pallas_call(
    _head_body,
    grid=(GRID,),
    in_specs=[
        pl.BlockSpec((BLK, H), lambda i: (i, 0)),
        pl.BlockSpec((BLK, H), lambda i: (i, 0)),
        pl.BlockSpec((BLK, 1), lambda i: (i, 0)),
        pl.BlockSpec((H, H), lambda i: (0, 0)),
        pl.BlockSpec((1, H), lambda i: (0, 0)),
        pl.BlockSpec((H, 8), lambda i: (0, 0)),
        pl.BlockSpec((1, 8), lambda i: (0, 0)),
    ],
    out_specs=pl.BlockSpec((BLK, 8), lambda i: (i, 0)),
    out_shape=jax.ShapeDtypeStruct((N_PAD, 8), jnp.float32),
)


def kernel(partial_charge, atomic_number, degree, ring_encoding, edge_index,
           W_in, b_in, W_att, a_src, a_dst, W1, b1, W_out, b_out):
    f32 = jnp.float32
    xin = jnp.concatenate(
        [partial_charge, atomic_number, degree, ring_encoding], axis=-1)
    xin = jnp.pad(xin.astype(f32), ((0, N_PAD - N), (0, 48 - 41)))
    win = jnp.pad(W_in.astype(f32), ((0, 48 - 41), (0, 0)))

    x, h, al = _pre_call(xin, win, b_in.reshape(1, H).astype(f32),
                         W_att.astype(f32), a_src.reshape(1, H).astype(f32),
                         a_dst.reshape(1, H).astype(f32))

    src = edge_index[0].astype(jnp.int32)
    dst = edge_index[1].astype(jnp.int32)
    agg, den, _as1, _ad1 = _edge_call(h, al, src, dst)

    out = _head_call(x, agg, den.reshape(N_PAD, 1),
                     W1.astype(f32), b1.reshape(1, H).astype(f32),
                     jnp.pad(W_out.astype(f32), ((0, 0), (0, 2))),
                     jnp.pad(b_out.astype(f32), (0, 2)).reshape(1, 8))
    return out[:N, :6]


# speedup vs baseline: 41.4736x; 1.4345x over previous
import jax
import jax.numpy as jnp
from jax import lax
from jax.experimental import pallas as pl
from jax.experimental.pallas import tpu as pltpu
from jax.experimental.pallas import tpu_sc as plsc

N = 100000
E = 1600000
H = 32
N_PAD = 102400
HALF = N_PAD // 2
BLK = 2048
GRID = N_PAD // BLK

NTILE = 16
EDGES_PER_TILE = E // NTILE
CHUNK = 400
NCHUNK = EDGES_PER_TILE // CHUNK
VREGS = CHUNK // 16
SEG = N_PAD // NTILE
ASTG = 640
DRAIN = HALF // NTILE


def _pre_body(xin_ref, win_ref, bin_ref, watt_ref, asr_ref, adr_ref,
              x_ref, h_ref, al_ref):
    z = jnp.dot(xin_ref[...], win_ref[...],
                preferred_element_type=jnp.float32) + bin_ref[...]
    x = jnp.where(z > 0, z, jnp.exp(z) - 1.0)
    x_ref[...] = x
    h = jnp.dot(x, watt_ref[...], preferred_element_type=jnp.float32)
    h_ref[...] = h
    a_s = lax.dot_general(asr_ref[...], h, (((1,), (1,)), ((), ())),
                          preferred_element_type=jnp.float32)
    a_d = lax.dot_general(adr_ref[...], h, (((1,), (1,)), ((), ())),
                          preferred_element_type=jnp.float32)
    al_ref[...] = jnp.concatenate(
        [jnp.broadcast_to(a_s, (8, BLK)), jnp.broadcast_to(a_d, (8, BLK))], 0)


_pre_call = pl.pallas_call(
    _pre_body,
    grid=(GRID,),
    in_specs=[
        pl.BlockSpec((BLK, 48), lambda i: (i, 0)),
        pl.BlockSpec((48, H), lambda i: (0, 0)),
        pl.BlockSpec((1, H), lambda i: (0, 0)),
        pl.BlockSpec((H, H), lambda i: (0, 0)),
        pl.BlockSpec((1, H), lambda i: (0, 0)),
        pl.BlockSpec((1, H), lambda i: (0, 0)),
    ],
    out_specs=[
        pl.BlockSpec((BLK, H), lambda i: (i, 0)),
        pl.BlockSpec((BLK, H), lambda i: (i, 0)),
        pl.BlockSpec((16, BLK), lambda i: (0, i)),
    ],
    out_shape=[
        jax.ShapeDtypeStruct((N_PAD, H), jnp.float32),
        jax.ShapeDtypeStruct((N_PAD, H), jnp.float32),
        jax.ShapeDtypeStruct((16, N_PAD), jnp.float32),
    ],
)


def _edge_body(h_hbm, al_hbm, src_hbm, dst_hbm,
               agg_hbm, den_hbm, as1_hbm, ad1_hbm,
               agg_sp, den_sp,
               srcv, dstv, asv, adv, pbuf, dlocv, rows, abuf, sem, sem2, sem3):
    c = lax.axis_index("c")
    s = lax.axis_index("s")
    lo = c * HALF

    z16 = jnp.zeros((16,), jnp.float32)

    def zr_body(i, _):
        rows[i, pl.ds(0, 16)] = z16
        rows[i, pl.ds(16, 16)] = z16
        return 0

    lax.fori_loop(0, CHUNK, zr_body, 0)

    def zp_body(i, _):
        pbuf[pl.ds(i * 16, 16)] = z16
        return 0

    lax.fori_loop(0, VREGS, zp_body, 0)

    db = s * DRAIN
    for q in range(DRAIN // CHUNK):
        pltpu.sync_copy(rows, agg_sp.at[pl.ds(db + q * CHUNK, CHUNK)])
        pltpu.sync_copy(pbuf, den_sp.at[pl.ds(db + q * CHUNK, CHUNK)])

    @pl.when(s == 0)
    def _zero_trash():
        pltpu.sync_copy(rows.at[pl.ds(0, 16)], agg_sp.at[pl.ds(HALF, 16)])
        pltpu.sync_copy(pbuf.at[pl.ds(0, 16)], den_sp.at[pl.ds(HALF, 16)])

    def al_body(q, _):
        b2 = s * SEG + q * ASTG
        pltpu.sync_copy(al_hbm.at[pl.ds(0, 8), pl.ds(b2, ASTG)], abuf)
        pltpu.sync_copy(abuf.at[0], as1_hbm.at[pl.ds(b2, ASTG)])
        pltpu.sync_copy(al_hbm.at[pl.ds(8, 8), pl.ds(b2, ASTG)], abuf)
        pltpu.sync_copy(abuf.at[0], ad1_hbm.at[pl.ds(b2, ASTG)])
        return 0

    lax.fori_loop(0, SEG // ASTG, al_body, 0)

    plsc.subcore_barrier()

    lane = lax.iota(jnp.int32, 16)

    def chunk_body(k, _):
        base = s * EDGES_PER_TILE + k * CHUNK
        d1 = pltpu.async_copy(src_hbm.at[pl.ds(base, CHUNK)], srcv, sem)
        d2 = pltpu.async_copy(dst_hbm.at[pl.ds(base, CHUNK)], dstv, sem2)
        d1.wait()
        d2.wait()
        g1 = pltpu.async_copy(as1_hbm.at[srcv], asv, sem)
        g2 = pltpu.async_copy(ad1_hbm.at[dstv], adv, sem2)
        g3 = pltpu.async_copy(h_hbm.at[srcv], rows, sem3)
        g1.wait()
        g2.wait()

        def vreg_body(j, _):
            o = j * 16
            a = asv[pl.ds(o, 16)] + adv[pl.ds(o, 16)]
            e = jnp.maximum(a, 0.2 * a)
            p = jnp.exp(e)
            dd = dstv[pl.ds(o, 16)]
            inhalf = (dd >= lo) & (dd < lo + HALF)
            pbuf[pl.ds(o, 16)] = jnp.where(inhalf, p, 0.0)
            dlocv[pl.ds(o, 16)] = jnp.where(inhalf, dd - lo, HALF + lane)
            return 0

        lax.fori_loop(0, VREGS, vreg_body, 0)
        g3.wait()

        def wbody(g, _):
            o = g * 16
            pv16 = pbuf[pl.ds(o, 16)]
            for u in range(16):
                ei = o + u
                pv = jnp.full((16,), pv16[u], jnp.float32)
                rows[ei, pl.ds(0, 16)] = rows[ei, pl.ds(0, 16)] * pv
                rows[ei, pl.ds(16, 16)] = rows[ei, pl.ds(16, 16)] * pv
            return 0

        lax.fori_loop(0, VREGS, wbody, 0)

        s1 = pltpu.async_copy(rows, agg_sp.at[dlocv], sem, add=True)
        s2 = pltpu.async_copy(pbuf, den_sp.at[dlocv], sem2, add=True)
        s1.wait()
        s2.wait()
        return 0

    lax.fori_loop(0, NCHUNK, chunk_body, 0)

    plsc.subcore_barrier()

    ob = lo + db
    for q in range(DRAIN // CHUNK):
        qq = q * CHUNK
        pltpu.sync_copy(agg_sp.at[pl.ds(db + qq, CHUNK)], rows)
        pltpu.sync_copy(rows, agg_hbm.at[pl.ds(ob + qq, CHUNK)])
        pltpu.sync_copy(den_sp.at[pl.ds(db + qq, CHUNK)], pbuf)
        pltpu.sync_copy(pbuf, den_hbm.at[pl.ds(ob + qq, CHUNK)])


_edge_call = pl.kernel(
    _edge_body,
    out_type=[
        jax.ShapeDtypeStruct((N_PAD, H), jnp.float32),
        jax.ShapeDtypeStruct((N_PAD,), jnp.float32),
        jax.ShapeDtypeStruct((N_PAD,), jnp.float32),
        jax.ShapeDtypeStruct((N_PAD,), jnp.float32),
    ],
    mesh=plsc.VectorSubcoreMesh(core_axis_name="c", subcore_axis_name="s"),
    compiler_params=pltpu.CompilerParams(use_tc_tiling_on_sc=False),
    scratch_types=[
        pltpu.VMEM_SHARED((HALF + 16, H), jnp.float32),
        pltpu.VMEM_SHARED((HALF + 16,), jnp.float32),
        pltpu.VMEM((CHUNK,), jnp.int32),
        pltpu.VMEM((CHUNK,), jnp.int32),
        pltpu.VMEM((CHUNK,), jnp.float32),
        pltpu.VMEM((CHUNK,), jnp.float32),
        pltpu.VMEM((CHUNK,), jnp.float32),
        pltpu.VMEM((CHUNK,), jnp.int32),
        pltpu.VMEM((CHUNK, H), jnp.float32),
        pltpu.VMEM((8, ASTG), jnp.float32),
        pltpu.SemaphoreType.DMA,
        pltpu.SemaphoreType.DMA,
        pltpu.SemaphoreType.DMA,
    ],
)


def _head_body(x_ref, agg_ref, den_ref, w1_ref, b1_ref, wo_ref, bo_ref, o_ref):
    hn = x_ref[...] + agg_ref[...] / (den_ref[...] + 1e-16)
    sa = jnp.where(hn > 0, hn, jnp.exp(hn) - 1.0)
    sb = jnp.dot(sa, w1_ref[...], preferred_element_type=jnp.float32) + b1_ref[...]
    sc = jnp.where(sb > 0, sb, jnp.exp(sb) - 1.0)
    o_ref[...] = jnp.dot(sc, wo_ref[...],
                         preferred_element_type=jnp.float32) + bo_ref[...]


_head_call = pl.pallas_call(
    _head_body,
    grid=(GRID,),
    in_specs=[
        pl.BlockSpec((BLK, H), lambda i: (i, 0)),
        pl.BlockSpec((BLK, H), lambda i: (i, 0)),
        pl.BlockSpec((BLK, 1), lambda i: (i, 0)),
        pl.BlockSpec((H, H), lambda i: (0, 0)),
        pl.BlockSpec((1, H), lambda i: (0, 0)),
        pl.BlockSpec((H, 8), lambda i: (0, 0)),
        pl.BlockSpec((1, 8), lambda i: (0, 0)),
    ],
    out_specs=pl.BlockSpec((BLK, 8), lambda i: (i, 0)),
    out_shape=jax.ShapeDtypeStruct((N_PAD, 8), jnp.float32),
)


def kernel(partial_charge, atomic_number, degree, ring_encoding, edge_index,
           W_in, b_in, W_att, a_src, a_dst, W1, b1, W_out, b_out):
    f32 = jnp.float32
    xin = jnp.concatenate(
        [partial_charge, atomic_number, degree, ring_encoding], axis=-1)
    xin = jnp.pad(xin.astype(f32), ((0, N_PAD - N), (0, 48 - 41)))
    win = jnp.pad(W_in.astype(f32), ((0, 48 - 41), (0, 0)))

    x, h, al = _pre_call(xin, win, b_in.reshape(1, H).astype(f32),
                         W_att.astype(f32), a_src.reshape(1, H).astype(f32),
                         a_dst.reshape(1, H).astype(f32))

    src = edge_index[0].astype(jnp.int32)
    dst = edge_index[1].astype(jnp.int32)
    agg, den, _as1, _ad1 = _edge_call(h, al, src, dst)

    out = _head_call(x, agg, den.reshape(N_PAD, 1),
                     W1.astype(f32), b1.reshape(1, H).astype(f32),
                     jnp.pad(W_out.astype(f32), ((0, 0), (0, 2))),
                     jnp.pad(b_out.astype(f32), (0, 2)).reshape(1, 8))
    return out[:N, :6]


# speedup vs baseline: 49.7881x; 1.2005x over previous
import jax
import jax.numpy as jnp
from jax import lax
from jax.experimental import pallas as pl
from jax.experimental.pallas import tpu as pltpu
from jax.experimental.pallas import tpu_sc as plsc

N = 100000
E = 1600000
H = 32
N_PAD = 102400
HALF = N_PAD // 2
BLK = 2048
GRID = N_PAD // BLK

NTILE = 16
EDGES_PER_TILE = E // NTILE
CHUNK = 400
NCHUNK = EDGES_PER_TILE // CHUNK
VREGS = CHUNK // 16
SEG = N_PAD // NTILE
ASTG = 640
DRAIN = HALF // NTILE


def _pre_body(xin_ref, win_ref, bin_ref, watt_ref, asr_ref, adr_ref,
              x_ref, h_ref, al_ref):
    z = jnp.dot(xin_ref[...], win_ref[...],
                preferred_element_type=jnp.float32) + bin_ref[...]
    x = jnp.where(z > 0, z, jnp.exp(z) - 1.0)
    x_ref[...] = x
    h = jnp.dot(x, watt_ref[...], preferred_element_type=jnp.float32)
    h_ref[...] = h
    a_s = lax.dot_general(asr_ref[...], h, (((1,), (1,)), ((), ())),
                          preferred_element_type=jnp.float32)
    a_d = lax.dot_general(adr_ref[...], h, (((1,), (1,)), ((), ())),
                          preferred_element_type=jnp.float32)
    al_ref[...] = jnp.concatenate(
        [jnp.broadcast_to(a_s, (8, BLK)), jnp.broadcast_to(a_d, (8, BLK))], 0)


_pre_call = pl.pallas_call(
    _pre_body,
    grid=(GRID,),
    in_specs=[
        pl.BlockSpec((BLK, 48), lambda i: (i, 0)),
        pl.BlockSpec((48, H), lambda i: (0, 0)),
        pl.BlockSpec((1, H), lambda i: (0, 0)),
        pl.BlockSpec((H, H), lambda i: (0, 0)),
        pl.BlockSpec((1, H), lambda i: (0, 0)),
        pl.BlockSpec((1, H), lambda i: (0, 0)),
    ],
    out_specs=[
        pl.BlockSpec((BLK, H), lambda i: (i, 0)),
        pl.BlockSpec((BLK, H), lambda i: (i, 0)),
        pl.BlockSpec((16, BLK), lambda i: (0, i)),
    ],
    out_shape=[
        jax.ShapeDtypeStruct((N_PAD, H), jnp.float32),
        jax.ShapeDtypeStruct((N_PAD, H), jnp.float32),
        jax.ShapeDtypeStruct((16, N_PAD), jnp.float32),
    ],
)


def _edge_body(h_hbm, al_hbm, src_hbm, dst_hbm,
               agg_hbm, den_hbm, as1_hbm, ad1_hbm,
               agg_sp, den_sp,
               srcv, dstv, asv, adv, pbuf, dlocv, rows, abuf,
               semS, semD, semAS, semAD, semG, semX, semY):
    c = lax.axis_index("c")
    s = lax.axis_index("s")
    lo = c * HALF

    z16 = jnp.zeros((16,), jnp.float32)

    def zr_body(i, _):
        rows[i, pl.ds(0, 16)] = z16
        rows[i, pl.ds(16, 16)] = z16
        return 0

    lax.fori_loop(0, CHUNK, zr_body, 0)

    def zp_body(i, _):
        pbuf[pl.ds(i * 16, 16)] = z16
        return 0

    lax.fori_loop(0, VREGS, zp_body, 0)

    db = s * DRAIN
    for q in range(DRAIN // CHUNK):
        pltpu.sync_copy(rows, agg_sp.at[pl.ds(db + q * CHUNK, CHUNK)])
        pltpu.sync_copy(pbuf, den_sp.at[pl.ds(db + q * CHUNK, CHUNK)])

    @pl.when(s == 0)
    def _zero_trash():
        pltpu.sync_copy(rows.at[pl.ds(0, 16)], agg_sp.at[pl.ds(HALF, 16)])
        pltpu.sync_copy(pbuf.at[pl.ds(0, 16)], den_sp.at[pl.ds(HALF, 16)])

    def al_body(q, _):
        b2 = s * SEG + q * ASTG
        pltpu.sync_copy(al_hbm.at[pl.ds(0, 8), pl.ds(b2, ASTG)], abuf)
        pltpu.sync_copy(abuf.at[0], as1_hbm.at[pl.ds(b2, ASTG)])
        pltpu.sync_copy(al_hbm.at[pl.ds(8, 8), pl.ds(b2, ASTG)], abuf)
        pltpu.sync_copy(abuf.at[0], ad1_hbm.at[pl.ds(b2, ASTG)])
        return 0

    lax.fori_loop(0, SEG // ASTG, al_body, 0)

    plsc.subcore_barrier()

    lane = lax.iota(jnp.int32, 16)

    def fire_sd(k, par):
        base = s * EDGES_PER_TILE + k * CHUNK
        pltpu.async_copy(src_hbm.at[pl.ds(base, CHUNK)], srcv.at[par], semS)
        pltpu.async_copy(dst_hbm.at[pl.ds(base, CHUNK)], dstv.at[par], semD)

    def drain_sd(k, par):
        base = s * EDGES_PER_TILE + k * CHUNK
        pltpu.make_async_copy(
            src_hbm.at[pl.ds(base, CHUNK)], srcv.at[par], semS).wait()
        pltpu.make_async_copy(
            dst_hbm.at[pl.ds(base, CHUNK)], dstv.at[par], semD).wait()

    def fire_al(par):
        pltpu.async_copy(as1_hbm.at[srcv.at[par]], asv.at[par], semAS)
        pltpu.async_copy(ad1_hbm.at[dstv.at[par]], adv.at[par], semAD)

    def drain_al(par):
        pltpu.make_async_copy(
            as1_hbm.at[srcv.at[par]], asv.at[par], semAS).wait()
        pltpu.make_async_copy(
            ad1_hbm.at[dstv.at[par]], adv.at[par], semAD).wait()

    fire_sd(0, 0)
    drain_sd(0, 0)
    fire_al(0)

    def chunk_body(k, _):
        par = k & 1
        nxt = 1 - par
        g3 = pltpu.async_copy(h_hbm.at[srcv.at[par]], rows, semG)

        @pl.when(k + 1 < NCHUNK)
        def _pf_sd():
            fire_sd(k + 1, nxt)

        drain_al(par)

        def vreg_body(j, _):
            o = j * 16
            a = asv[par, pl.ds(o, 16)] + adv[par, pl.ds(o, 16)]
            e = jnp.maximum(a, 0.2 * a)
            p = jnp.exp(e)
            dd = dstv[par, pl.ds(o, 16)]
            inhalf = (dd >= lo) & (dd < lo + HALF)
            pbuf[pl.ds(o, 16)] = jnp.where(inhalf, p, 0.0)
            dlocv[pl.ds(o, 16)] = jnp.where(inhalf, dd - lo, HALF + lane)
            return 0

        lax.fori_loop(0, VREGS, vreg_body, 0)
        g3.wait()

        def wbody(g, _):
            o = g * 16
            pv16 = pbuf[pl.ds(o, 16)]
            for u in range(16):
                ei = o + u
                pv = jnp.full((16,), pv16[u], jnp.float32)
                rows[ei, pl.ds(0, 16)] = rows[ei, pl.ds(0, 16)] * pv
                rows[ei, pl.ds(16, 16)] = rows[ei, pl.ds(16, 16)] * pv
            return 0

        lax.fori_loop(0, VREGS, wbody, 0)

        @pl.when(k + 1 < NCHUNK)
        def _pf_al():
            drain_sd(k + 1, nxt)
            fire_al(nxt)

        s1 = pltpu.async_copy(rows, agg_sp.at[dlocv], semX, add=True)
        s2 = pltpu.async_copy(pbuf, den_sp.at[dlocv], semY, add=True)
        s1.wait()
        s2.wait()
        return 0

    lax.fori_loop(0, NCHUNK, chunk_body, 0)

    plsc.subcore_barrier()

    ob = lo + db
    for q in range(DRAIN // CHUNK):
        qq = q * CHUNK
        pltpu.sync_copy(agg_sp.at[pl.ds(db + qq, CHUNK)], rows)
        pltpu.sync_copy(rows, agg_hbm.at[pl.ds(ob + qq, CHUNK)])
        pltpu.sync_copy(den_sp.at[pl.ds(db + qq, CHUNK)], pbuf)
        pltpu.sync_copy(pbuf, den_hbm.at[pl.ds(ob + qq, CHUNK)])


_edge_call = pl.kernel(
    _edge_body,
    out_type=[
        jax.ShapeDtypeStruct((N_PAD, H), jnp.float32),
        jax.ShapeDtypeStruct((N_PAD,), jnp.float32),
        jax.ShapeDtypeStruct((N_PAD,), jnp.float32),
        jax.ShapeDtypeStruct((N_PAD,), jnp.float32),
    ],
    mesh=plsc.VectorSubcoreMesh(core_axis_name="c", subcore_axis_name="s"),
    compiler_params=pltpu.CompilerParams(use_tc_tiling_on_sc=False),
    scratch_types=[
        pltpu.VMEM_SHARED((HALF + 16, H), jnp.float32),
        pltpu.VMEM_SHARED((HALF + 16,), jnp.float32),
        pltpu.VMEM((2, CHUNK), jnp.int32),
        pltpu.VMEM((2, CHUNK), jnp.int32),
        pltpu.VMEM((2, CHUNK), jnp.float32),
        pltpu.VMEM((2, CHUNK), jnp.float32),
        pltpu.VMEM((CHUNK,), jnp.float32),
        pltpu.VMEM((CHUNK,), jnp.int32),
        pltpu.VMEM((CHUNK, H), jnp.float32),
        pltpu.VMEM((8, ASTG), jnp.float32),
        pltpu.SemaphoreType.DMA,
        pltpu.SemaphoreType.DMA,
        pltpu.SemaphoreType.DMA,
        pltpu.SemaphoreType.DMA,
        pltpu.SemaphoreType.DMA,
        pltpu.SemaphoreType.DMA,
        pltpu.SemaphoreType.DMA,
    ],
)


def _head_body(x_ref, agg_ref, den_ref, w1_ref, b1_ref, wo_ref, bo_ref, o_ref):
    hn = x_ref[...] + agg_ref[...] / (den_ref[...] + 1e-16)
    sa = jnp.where(hn > 0, hn, jnp.exp(hn) - 1.0)
    sb = jnp.dot(sa, w1_ref[...], preferred_element_type=jnp.float32) + b1_ref[...]
    sc = jnp.where(sb > 0, sb, jnp.exp(sb) - 1.0)
    o_ref[...] = jnp.dot(sc, wo_ref[...],
                         preferred_element_type=jnp.float32) + bo_ref[...]


_head_call = pl.pallas_call(
    _head_body,
    grid=(GRID,),
    in_specs=[
        pl.BlockSpec((BLK, H), lambda i: (i, 0)),
        pl.BlockSpec((BLK, H), lambda i: (i, 0)),
        pl.BlockSpec((BLK, 1), lambda i: (i, 0)),
        pl.BlockSpec((H, H), lambda i: (0, 0)),
        pl.BlockSpec((1, H), lambda i: (0, 0)),
        pl.BlockSpec((H, 8), lambda i: (0, 0)),
        pl.BlockSpec((1, 8), lambda i: (0, 0)),
    ],
    out_specs=pl.BlockSpec((BLK, 8), lambda i: (i, 0)),
    out_shape=jax.ShapeDtypeStruct((N_PAD, 8), jnp.float32),
)


def kernel(partial_charge, atomic_number, degree, ring_encoding, edge_index,
           W_in, b_in, W_att, a_src, a_dst, W1, b1, W_out, b_out):
    f32 = jnp.float32
    xin = jnp.concatenate(
        [partial_charge, atomic_number, degree, ring_encoding], axis=-1)
    xin = jnp.pad(xin.astype(f32), ((0, N_PAD - N), (0, 48 - 41)))
    win = jnp.pad(W_in.astype(f32), ((0, 48 - 41), (0, 0)))

    x, h, al = _pre_call(xin, win, b_in.reshape(1, H).astype(f32),
                         W_att.astype(f32), a_src.reshape(1, H).astype(f32),
                         a_dst.reshape(1, H).astype(f32))

    src = edge_index[0].astype(jnp.int32)
    dst = edge_index[1].astype(jnp.int32)
    agg, den, _as1, _ad1 = _edge_call(h, al, src, dst)

    out = _head_call(x, agg, den.reshape(N_PAD, 1),
                     W1.astype(f32), b1.reshape(1, H).astype(f32),
                     jnp.pad(W_out.astype(f32), ((0, 0), (0, 2))),
                     jnp.pad(b_out.astype(f32), (0, 2)).reshape(1, 8))
    return out[:N, :6]


# speedup vs baseline: 51.0910x; 1.0262x over previous
import jax
import jax.numpy as jnp
from jax import lax
from jax.experimental import pallas as pl
from jax.experimental.pallas import tpu as pltpu
from jax.experimental.pallas import tpu_sc as plsc

N = 100000
E = 1600000
H = 32
N_PAD = 102400
HALF = N_PAD // 2
BLK = 2048
GRID = N_PAD // BLK

NTILE = 16
EDGES_PER_TILE = E // NTILE
CHUNK = 400
NCHUNK = EDGES_PER_TILE // CHUNK
VREGS = CHUNK // 16
HC1 = 208
HC2 = CHUNK - HC1
SEG = N_PAD // NTILE
ASTG = 640
DRAIN = HALF // NTILE


def _pre_body(xin_ref, win_ref, bin_ref, watt_ref, asr_ref, adr_ref,
              x_ref, h_ref, al_ref):
    z = jnp.dot(xin_ref[...], win_ref[...],
                preferred_element_type=jnp.float32) + bin_ref[...]
    x = jnp.where(z > 0, z, jnp.exp(z) - 1.0)
    x_ref[...] = x
    h = jnp.dot(x, watt_ref[...], preferred_element_type=jnp.float32)
    h_ref[...] = h
    a_s = lax.dot_general(asr_ref[...], h, (((1,), (1,)), ((), ())),
                          preferred_element_type=jnp.float32)
    a_d = lax.dot_general(adr_ref[...], h, (((1,), (1,)), ((), ())),
                          preferred_element_type=jnp.float32)
    al_ref[...] = jnp.concatenate(
        [jnp.broadcast_to(a_s, (8, BLK)), jnp.broadcast_to(a_d, (8, BLK))], 0)


_pre_call = pl.pallas_call(
    _pre_body,
    grid=(GRID,),
    in_specs=[
        pl.BlockSpec((BLK, 48), lambda i: (i, 0)),
        pl.BlockSpec((48, H), lambda i: (0, 0)),
        pl.BlockSpec((1, H), lambda i: (0, 0)),
        pl.BlockSpec((H, H), lambda i: (0, 0)),
        pl.BlockSpec((1, H), lambda i: (0, 0)),
        pl.BlockSpec((1, H), lambda i: (0, 0)),
    ],
    out_specs=[
        pl.BlockSpec((BLK, H), lambda i: (i, 0)),
        pl.BlockSpec((BLK, H), lambda i: (i, 0)),
        pl.BlockSpec((16, BLK), lambda i: (0, i)),
    ],
    out_shape=[
        jax.ShapeDtypeStruct((N_PAD, H), jnp.float32),
        jax.ShapeDtypeStruct((N_PAD, H), jnp.float32),
        jax.ShapeDtypeStruct((16, N_PAD), jnp.float32),
    ],
)


def _edge_body(h_hbm, al_hbm, src_hbm, dst_hbm,
               agg_hbm, den_hbm, as1_hbm, ad1_hbm,
               agg_sp, den_sp,
               srcv, dstv, asv, adv, pbuf, pbufA, pbufB, dlocA, dlocB,
               rows, abuf,
               semS, semD, semAS, semAD, semG, semG2, semX, semY, semX2, semY2):
    c = lax.axis_index("c")
    s = lax.axis_index("s")
    lo = c * HALF

    z16 = jnp.zeros((16,), jnp.float32)

    def zr_body(i, _):
        rows[i, pl.ds(0, 16)] = z16
        rows[i, pl.ds(16, 16)] = z16
        return 0

    lax.fori_loop(0, CHUNK, zr_body, 0)

    def zp_body(i, _):
        pbuf[pl.ds(i * 16, 16)] = z16
        return 0

    lax.fori_loop(0, VREGS, zp_body, 0)

    db = s * DRAIN
    for q in range(DRAIN // CHUNK):
        pltpu.sync_copy(rows, agg_sp.at[pl.ds(db + q * CHUNK, CHUNK)])
        pltpu.sync_copy(pbuf, den_sp.at[pl.ds(db + q * CHUNK, CHUNK)])

    @pl.when(s == 0)
    def _zero_trash():
        pltpu.sync_copy(rows.at[pl.ds(0, 16)], agg_sp.at[pl.ds(HALF, 16)])
        pltpu.sync_copy(pbuf.at[pl.ds(0, 16)], den_sp.at[pl.ds(HALF, 16)])

    def al_body(q, _):
        b2 = s * SEG + q * ASTG
        pltpu.sync_copy(al_hbm.at[pl.ds(0, 8), pl.ds(b2, ASTG)], abuf)
        pltpu.sync_copy(abuf.at[0], as1_hbm.at[pl.ds(b2, ASTG)])
        pltpu.sync_copy(al_hbm.at[pl.ds(8, 8), pl.ds(b2, ASTG)], abuf)
        pltpu.sync_copy(abuf.at[0], ad1_hbm.at[pl.ds(b2, ASTG)])
        return 0

    lax.fori_loop(0, SEG // ASTG, al_body, 0)

    plsc.subcore_barrier()

    lane = lax.iota(jnp.int32, 16)

    def fire_sd(k, par):
        base = s * EDGES_PER_TILE + k * CHUNK
        pltpu.async_copy(src_hbm.at[pl.ds(base, CHUNK)], srcv.at[par], semS)
        pltpu.async_copy(dst_hbm.at[pl.ds(base, CHUNK)], dstv.at[par], semD)

    def drain_sd(k, par):
        base = s * EDGES_PER_TILE + k * CHUNK
        pltpu.make_async_copy(
            src_hbm.at[pl.ds(base, CHUNK)], srcv.at[par], semS).wait()
        pltpu.make_async_copy(
            dst_hbm.at[pl.ds(base, CHUNK)], dstv.at[par], semD).wait()

    def fire_al(par):
        pltpu.async_copy(as1_hbm.at[srcv.at[par]], asv.at[par], semAS)
        pltpu.async_copy(ad1_hbm.at[dstv.at[par]], adv.at[par], semAD)

    def drain_al(par):
        pltpu.make_async_copy(
            as1_hbm.at[srcv.at[par]], asv.at[par], semAS).wait()
        pltpu.make_async_copy(
            ad1_hbm.at[dstv.at[par]], adv.at[par], semAD).wait()

    fire_sd(0, 0)
    drain_sd(0, 0)
    fire_al(0)

    def chunk_body(k, _):
        par = k & 1
        nxt = 1 - par
        g3a = pltpu.async_copy(h_hbm.at[srcv.at[par, pl.ds(0, HC1)]],
                               rows.at[pl.ds(0, HC1)], semG)
        g3b = pltpu.async_copy(h_hbm.at[srcv.at[par, pl.ds(HC1, HC2)]],
                               rows.at[pl.ds(HC1, HC2)], semG2)

        @pl.when(k + 1 < NCHUNK)
        def _pf_sd():
            fire_sd(k + 1, nxt)

        drain_al(par)

        def vreg_a(j, _):
            o = j * 16
            a = asv[par, pl.ds(o, 16)] + adv[par, pl.ds(o, 16)]
            e = jnp.maximum(a, 0.2 * a)
            p = jnp.exp(e)
            dd = dstv[par, pl.ds(o, 16)]
            inhalf = (dd >= lo) & (dd < lo + HALF)
            pbufA[pl.ds(o, 16)] = jnp.where(inhalf, p, 0.0)
            dlocA[pl.ds(o, 16)] = jnp.where(inhalf, dd - lo, HALF + lane)
            return 0

        def vreg_b(j, _):
            o = j * 16
            oo = HC1 + o
            a = asv[par, pl.ds(oo, 16)] + adv[par, pl.ds(oo, 16)]
            e = jnp.maximum(a, 0.2 * a)
            p = jnp.exp(e)
            dd = dstv[par, pl.ds(oo, 16)]
            inhalf = (dd >= lo) & (dd < lo + HALF)
            pbufB[pl.ds(o, 16)] = jnp.where(inhalf, p, 0.0)
            dlocB[pl.ds(o, 16)] = jnp.where(inhalf, dd - lo, HALF + lane)
            return 0

        lax.fori_loop(0, HC1 // 16, vreg_a, 0)
        lax.fori_loop(0, HC2 // 16, vreg_b, 0)
        g3a.wait()

        def wbody_a(g, _):
            o = g * 16
            pv16 = pbufA[pl.ds(o, 16)]
            for u in range(16):
                ei = o + u
                pv = jnp.full((16,), pv16[u], jnp.float32)
                rows[ei, pl.ds(0, 16)] = rows[ei, pl.ds(0, 16)] * pv
                rows[ei, pl.ds(16, 16)] = rows[ei, pl.ds(16, 16)] * pv
            return 0

        lax.fori_loop(0, HC1 // 16, wbody_a, 0)
        s1a = pltpu.async_copy(rows.at[pl.ds(0, HC1)], agg_sp.at[dlocA],
                               semX, add=True)
        s2a = pltpu.async_copy(pbufA, den_sp.at[dlocA], semY, add=True)

        g3b.wait()

        def wbody_b(g, _):
            o = g * 16
            pv16 = pbufB[pl.ds(o, 16)]
            for u in range(16):
                ei = HC1 + o + u
                pv = jnp.full((16,), pv16[u], jnp.float32)
                rows[ei, pl.ds(0, 16)] = rows[ei, pl.ds(0, 16)] * pv
                rows[ei, pl.ds(16, 16)] = rows[ei, pl.ds(16, 16)] * pv
            return 0

        lax.fori_loop(0, HC2 // 16, wbody_b, 0)

        @pl.when(k + 1 < NCHUNK)
        def _pf_al():
            drain_sd(k + 1, nxt)
            fire_al(nxt)

        s1b = pltpu.async_copy(rows.at[pl.ds(HC1, HC2)], agg_sp.at[dlocB],
                               semX2, add=True)
        s2b = pltpu.async_copy(pbufB, den_sp.at[dlocB], semY2, add=True)
        s1a.wait()
        s2a.wait()
        s1b.wait()
        s2b.wait()
        return 0

    lax.fori_loop(0, NCHUNK, chunk_body, 0)

    plsc.subcore_barrier()

    ob = lo + db
    for q in range(DRAIN // CHUNK):
        qq = q * CHUNK
        pltpu.sync_copy(agg_sp.at[pl.ds(db + qq, CHUNK)], rows)
        pltpu.sync_copy(rows, agg_hbm.at[pl.ds(ob + qq, CHUNK)])
        pltpu.sync_copy(den_sp.at[pl.ds(db + qq, CHUNK)], pbuf)
        pltpu.sync_copy(pbuf, den_hbm.at[pl.ds(ob + qq, CHUNK)])


_edge_call = pl.kernel(
    _edge_body,
    out_type=[
        jax.ShapeDtypeStruct((N_PAD, H), jnp.float32),
        jax.ShapeDtypeStruct((N_PAD,), jnp.float32),
        jax.ShapeDtypeStruct((N_PAD,), jnp.float32),
        jax.ShapeDtypeStruct((N_PAD,), jnp.float32),
    ],
    mesh=plsc.VectorSubcoreMesh(core_axis_name="c", subcore_axis_name="s"),
    compiler_params=pltpu.CompilerParams(use_tc_tiling_on_sc=False),
    scratch_types=[
        pltpu.VMEM_SHARED((HALF + 16, H), jnp.float32),
        pltpu.VMEM_SHARED((HALF + 16,), jnp.float32),
        pltpu.VMEM((2, CHUNK), jnp.int32),
        pltpu.VMEM((2, CHUNK), jnp.int32),
        pltpu.VMEM((2, CHUNK), jnp.float32),
        pltpu.VMEM((2, CHUNK), jnp.float32),
        pltpu.VMEM((CHUNK,), jnp.float32),
        pltpu.VMEM((HC1,), jnp.float32),
        pltpu.VMEM((HC2,), jnp.float32),
        pltpu.VMEM((HC1,), jnp.int32),
        pltpu.VMEM((HC2,), jnp.int32),
        pltpu.VMEM((CHUNK, H), jnp.float32),
        pltpu.VMEM((8, ASTG), jnp.float32),
        pltpu.SemaphoreType.DMA,
        pltpu.SemaphoreType.DMA,
        pltpu.SemaphoreType.DMA,
        pltpu.SemaphoreType.DMA,
        pltpu.SemaphoreType.DMA,
        pltpu.SemaphoreType.DMA,
        pltpu.SemaphoreType.DMA,
        pltpu.SemaphoreType.DMA,
        pltpu.SemaphoreType.DMA,
        pltpu.SemaphoreType.DMA,
    ],
)


def _head_body(x_ref, agg_ref, den_ref, w1_ref, b1_ref, wo_ref, bo_ref, o_ref):
    hn = x_ref[...] + agg_ref[...] / (den_ref[...] + 1e-16)
    sa = jnp.where(hn > 0, hn, jnp.exp(hn) - 1.0)
    sb = jnp.dot(sa, w1_ref[...], preferred_element_type=jnp.float32) + b1_ref[...]
    sc = jnp.where(sb > 0, sb, jnp.exp(sb) - 1.0)
    o_ref[...] = jnp.dot(sc, wo_ref[...],
                         preferred_element_type=jnp.float32) + bo_ref[...]


_head_call = pl.pallas_call(
    _head_body,
    grid=(GRID,),
    in_specs=[
        pl.BlockSpec((BLK, H), lambda i: (i, 0)),
        pl.BlockSpec((BLK, H), lambda i: (i, 0)),
        pl.BlockSpec((BLK, 1), lambda i: (i, 0)),
        pl.BlockSpec((H, H), lambda i: (0, 0)),
        pl.BlockSpec((1, H), lambda i: (0, 0)),
        pl.BlockSpec((H, 8), lambda i: (0, 0)),
        pl.BlockSpec((1, 8), lambda i: (0, 0)),
    ],
    out_specs=pl.BlockSpec((BLK, 8), lambda i: (i, 0)),
    out_shape=jax.ShapeDtypeStruct((N_PAD, 8), jnp.float32),
)


def kernel(partial_charge, atomic_number, degree, ring_encoding, edge_index,
           W_in, b_in, W_att, a_src, a_dst, W1, b1, W_out, b_out):
    f32 = jnp.float32
    xin = jnp.concatenate(
        [partial_charge, atomic_number, degree, ring_encoding], axis=-1)
    xin = jnp.pad(xin.astype(f32), ((0, N_PAD - N), (0, 48 - 41)))
    win = jnp.pad(W_in.astype(f32), ((0, 48 - 41), (0, 0)))

    x, h, al = _pre_call(xin, win, b_in.reshape(1, H).astype(f32),
                         W_att.astype(f32), a_src.reshape(1, H).astype(f32),
                         a_dst.reshape(1, H).astype(f32))

    src = edge_index[0].astype(jnp.int32)
    dst = edge_index[1].astype(jnp.int32)
    agg, den, _as1, _ad1 = _edge_call(h, al, src, dst)

    out = _head_call(x, agg, den.reshape(N_PAD, 1),
                     W1.astype(f32), b1.reshape(1, H).astype(f32),
                     jnp.pad(W_out.astype(f32), ((0, 0), (0, 2))),
                     jnp.pad(b_out.astype(f32), (0, 2)).reshape(1, 8))
    return out[:N, :6]
